# baseline (speedup 1.0000x reference)
import functools

import jax
import jax.numpy as jnp
from jax import lax
from jax.experimental import pallas as pl
from jax.experimental.pallas import tpu as pltpu
from jax.experimental.pallas import tpu_sc as plsc

T = 16384
E = 256
K = 8
NGROUP = 8
GSIZE = E // NGROUP
NVREG = E // 16
NC, NS, L = 2, 16, 16
NW = NC * NS
TPW = T // NW
CHUNK = 128
NBLK = CHUNK // L
NCHUNK = TPW // CHUNK
NEG_INF = float("-inf")


def _routing_body(lf_hbm, bias_hbm, oi_hbm, ov_hbm,
                  lbuf, bias_v, swb_scr, tau_scr, sel_scr, stage_scr,
                  oi_scr, ov_scr):
    wid = lax.axis_index("s") * NC + lax.axis_index("c")
    iota = lax.iota(jnp.int32, L)
    iota_e = iota * E
    pltpu.sync_copy(bias_hbm, bias_v)
    stage_scr[pl.ds(0, L)] = iota
    tok0 = wid * TPW

    @pl.loop(0, NCHUNK)
    def _chunk(ci):
        base_tok = tok0 + ci * CHUNK
        pltpu.sync_copy(lf_hbm.at[pl.ds(base_tok * E, CHUNK * E)], lbuf)

        @pl.loop(0, CHUNK)
        def _pre(t):
            toff = t * E
            for j in range(NVREG):
                x = lbuf[pl.ds(toff + j * L, L)]
                s = 1.0 / (1.0 + jnp.exp(-x))
                swb_scr[pl.ds(toff + j * L, L)] = s + bias_v[pl.ds(j * L, L)]

        @pl.loop(0, NBLK)
        def _blk(tb):
            gidx = iota_e + tb * (L * E)
            gs = []
            for g in range(NGROUP):
                m1 = jnp.full((L,), NEG_INF, jnp.float32)
                m2 = m1
                for j in range(GSIZE):
                    w = plsc.load_gather(swb_scr, [gidx + (g * GSIZE + j)])
                    m2 = jnp.maximum(m2, jnp.minimum(m1, w))
                    m1 = jnp.maximum(m1, w)
                gs.append(m1 + m2)
            sel_acc = [None] * NGROUP
            for r in range(4):
                m = gs[0]
                for g in range(1, NGROUP):
                    m = jnp.maximum(m, gs[g])
                found = jnp.zeros((L,), jnp.bool_)
                for g in range(NGROUP):
                    is_arg = (gs[g] == m) & ~found
                    found = found | is_arg
                    sel_acc[g] = is_arg if r == 0 else sel_acc[g] | is_arg
                    gs[g] = jnp.where(is_arg, NEG_INF, gs[g])
            for g in range(NGROUP):
                sel_scr[pl.ds(g * CHUNK + tb * L, L)] = jnp.where(
                    sel_acc[g], 1.0, 0.0
                )
            r_init = tuple(jnp.full((L,), NEG_INF, jnp.float32) for _ in range(K))

            @pl.loop(0, NGROUP, init_carry=r_init)
            def _tau(g, rs):
                rs = list(rs)
                selg = sel_scr[pl.ds(g * CHUNK + tb * L, L)] > 0.5
                g0 = g * GSIZE
                for j in range(GSIZE):
                    w = plsc.load_gather(swb_scr, [gidx + (g0 + j)])
                    wm = jnp.where(selg, w, NEG_INF)
                    for i in range(K - 1, 0, -1):
                        rs[i] = jnp.maximum(rs[i], jnp.minimum(rs[i - 1], wm))
                    rs[0] = jnp.maximum(rs[0], wm)
                return tuple(rs)

            tau_scr[pl.ds(tb * L, L)] = _tau[K - 1]

        @pl.loop(0, CHUNK)
        def _tok(t):
            toff = t * E
            tauv = plsc.load_gather(tau_scr, [jnp.broadcast_to(t, (L,))])
            o = 0
            for g in range(NGROUP):
                selv = plsc.load_gather(
                    sel_scr, [jnp.broadcast_to(g * CHUNK + t, (L,))]
                )
                keep_g = selv > 0.5
                for h in range(2):
                    j = 2 * g + h
                    w = swb_scr[pl.ds(toff + j * L, L)]
                    m = (w >= tauv) & keep_g
                    plsc.store_compressed(
                        stage_scr.at[pl.ds(o, L)], iota + j * L, mask=m
                    )
                    cnt = plsc.all_reduce_population_count(m)
                    o = o + cnt[0]
            ids16 = stage_scr[pl.ds(0, L)]
            mask8 = iota < K
            lg = plsc.load_gather(lbuf, [ids16 + toff])
            s16 = 1.0 / (1.0 + jnp.exp(-lg))
            s8 = jnp.where(mask8, s16, 0.0)
            denom = jnp.broadcast_to(jnp.sum(s8) + 1e-20, (L,))
            vals = s8 * 2.5 / denom
            keys = jnp.where(mask8, vals, -1.0)
            ok, oi = plsc.sort_key_val(keys, ids16, descending=True)
            plsc.store_compressed(ov_scr.at[pl.ds(t * K, L)], ok, mask=mask8)
            plsc.store_compressed(oi_scr.at[pl.ds(t * K, L)], oi, mask=mask8)

        pltpu.sync_copy(ov_scr.at[pl.ds(0, CHUNK * K)],
                        ov_hbm.at[pl.ds(base_tok * K, CHUNK * K)])
        pltpu.sync_copy(oi_scr.at[pl.ds(0, CHUNK * K)],
                        oi_hbm.at[pl.ds(base_tok * K, CHUNK * K)])


_router = functools.partial(
    pl.kernel,
    out_type=(
        jax.ShapeDtypeStruct((T * K,), jnp.int32),
        jax.ShapeDtypeStruct((T * K,), jnp.float32),
    ),
    mesh=plsc.VectorSubcoreMesh(
        core_axis_name="c", subcore_axis_name="s", num_cores=NC, num_subcores=NS
    ),
    compiler_params=pltpu.CompilerParams(needs_layout_passes=False),
    scratch_types=[
        pltpu.VMEM((CHUNK * E,), jnp.float32),
        pltpu.VMEM((E,), jnp.float32),
        pltpu.VMEM((CHUNK * E,), jnp.float32),
        pltpu.VMEM((CHUNK,), jnp.float32),
        pltpu.VMEM((NGROUP * CHUNK,), jnp.float32),
        pltpu.VMEM((E + L,), jnp.int32),
        pltpu.VMEM((CHUNK * K + K,), jnp.int32),
        pltpu.VMEM((CHUNK * K + K,), jnp.float32),
    ],
)(_routing_body)


def kernel(logits, e_score_correction_bias):
    oi, ov = _router(logits.reshape(-1), e_score_correction_bias)
    return oi.reshape(T, K), ov.reshape(T, K)


# speedup vs baseline: 1.3314x; 1.3314x over previous
import functools

import jax
import jax.numpy as jnp
from jax import lax
from jax.experimental import pallas as pl
from jax.experimental.pallas import tpu as pltpu
from jax.experimental.pallas import tpu_sc as plsc

T = 16384
E = 256
K = 8
NGROUP = 8
GSIZE = E // NGROUP
NC, NS, L = 2, 16, 16
NW = NC * NS
TPW = T // NW
CHUNK = 128
NCHUNK = TPW // CHUNK
NEG_INF = float("-inf")


def _merge_top16(ka, va, kb, vb):
    kr = lax.rev(kb, (0,))
    vr = lax.rev(vb, (0,))
    ge = ka >= kr
    hk = jnp.where(ge, ka, kr)
    hv = jnp.where(ge, va, vr)
    return plsc.sort_key_val(hk, hv, descending=True)


def _routing_body(lf_hbm, bias_hbm, oi_hbm, ov_hbm,
                  lbuf, bias_v, sc_scr, hk_scr, hid_scr, lo_scr,
                  oi_scr, ov_scr):
    wid = lax.axis_index("s") * NC + lax.axis_index("c")
    iota = lax.iota(jnp.int32, L)
    pltpu.sync_copy(bias_hbm, bias_v)
    tok0 = wid * TPW

    @pl.loop(0, NCHUNK)
    def _chunk(ci):
        base_tok = tok0 + ci * CHUNK
        pltpu.sync_copy(lf_hbm.at[pl.ds(base_tok * E, CHUNK * E)], lbuf)

        @pl.loop(0, CHUNK)
        def _tok(t):
            toff = t * E
            gv = jnp.full((L,), NEG_INF, jnp.float32)
            for g in range(NGROUP):
                xa = lbuf[pl.ds(toff + g * GSIZE, L)]
                xb = lbuf[pl.ds(toff + g * GSIZE + L, L)]
                sa = 1.0 / (1.0 + jnp.exp(-xa))
                sb = 1.0 / (1.0 + jnp.exp(-xb))
                sc_scr[pl.ds(g * GSIZE, L)] = sa
                sc_scr[pl.ds(g * GSIZE + L, L)] = sb
                a = sa + bias_v[pl.ds(g * GSIZE, L)]
                b = sb + bias_v[pl.ds(g * GSIZE + L, L)]
                ge = a >= b
                hi = jnp.where(ge, a, b)
                lo = jnp.where(ge, b, a)
                hi_src = jnp.where(ge, g * GSIZE + iota, g * GSIZE + L + iota)
                hk, hid = plsc.sort_key_val(hi, hi_src, descending=True)
                hk_scr[pl.ds(g * L, L)] = hk
                hid_scr[pl.ds(g * L, L)] = hid
                lo_scr[pl.ds(g * L, L)] = lo
                psm = jnp.max(a + b)
                s = jnp.maximum(hk[0] + hk[1], psm)
                gv = jnp.where(iota == g, s, gv)
            _, gid = plsc.sort_key_val(gv, iota, descending=True)
            kept = []
            for r in range(4):
                base = gid[r] * L
                hk_r = hk_scr[pl.ds(base, L)]
                hid_r = hid_scr[pl.ds(base, L)]
                hid_sh = plsc.load_gather(hid_scr, [base + ((iota - 8) & (L - 1))])
                glo_sh = plsc.load_gather(lo_scr, [base + (hid_sh & (L - 1))])
                mlow = iota < 8
                cw = jnp.where(mlow, hk_r, glo_sh)
                cid = jnp.where(mlow, hid_r, hid_sh ^ L)
                kept.append(plsc.sort_key_val(cw, cid, descending=True))
            u0 = _merge_top16(*kept[0], *kept[1])
            u1 = _merge_top16(*kept[2], *kept[3])
            fk, fv = _merge_top16(*u0, *u1)
            mask8 = iota < K
            sgath = plsc.load_gather(sc_scr, [fv])
            s8 = jnp.where(mask8, sgath, 0.0)
            denom = jnp.broadcast_to(jnp.sum(s8) + 1e-20, (L,))
            vals = s8 * 2.5 / denom
            keys = jnp.where(mask8, vals, -1.0)
            ok, oi = plsc.sort_key_val(keys, fv, descending=True)
            plsc.store_compressed(ov_scr.at[pl.ds(t * K, L)], ok, mask=mask8)
            plsc.store_compressed(oi_scr.at[pl.ds(t * K, L)], oi, mask=mask8)

        pltpu.sync_copy(ov_scr.at[pl.ds(0, CHUNK * K)],
                        ov_hbm.at[pl.ds(base_tok * K, CHUNK * K)])
        pltpu.sync_copy(oi_scr.at[pl.ds(0, CHUNK * K)],
                        oi_hbm.at[pl.ds(base_tok * K, CHUNK * K)])


_router = functools.partial(
    pl.kernel,
    out_type=(
        jax.ShapeDtypeStruct((T * K,), jnp.int32),
        jax.ShapeDtypeStruct((T * K,), jnp.float32),
    ),
    mesh=plsc.VectorSubcoreMesh(
        core_axis_name="c", subcore_axis_name="s", num_cores=NC, num_subcores=NS
    ),
    compiler_params=pltpu.CompilerParams(needs_layout_passes=False),
    scratch_types=[
        pltpu.VMEM((CHUNK * E,), jnp.float32),
        pltpu.VMEM((E,), jnp.float32),
        pltpu.VMEM((E,), jnp.float32),
        pltpu.VMEM((NGROUP * L,), jnp.float32),
        pltpu.VMEM((NGROUP * L,), jnp.int32),
        pltpu.VMEM((NGROUP * L,), jnp.float32),
        pltpu.VMEM((CHUNK * K + K,), jnp.int32),
        pltpu.VMEM((CHUNK * K + K,), jnp.float32),
    ],
)(_routing_body)


def kernel(logits, e_score_correction_bias):
    oi, ov = _router(logits.reshape(-1), e_score_correction_bias)
    return oi.reshape(T, K), ov.reshape(T, K)
